# Initial kernel scaffold; baseline (speedup 1.0000x reference)
import jax
import jax.numpy as jnp
from jax.experimental import pallas as pl


def kernel(input, word2syllable, embedding):
    raise NotImplementedError("write your pallas kernel here")


# speedup vs baseline: 4.5522x; 4.5522x over previous
import functools

import jax
import jax.numpy as jnp
from jax import lax
from jax.experimental import pallas as pl
from jax.experimental.pallas import tpu as pltpu
from jax.experimental.pallas import tpu_sc as plsc

NC = 2
NS = 16
NW = NC * NS
L = 16

EMB_DIM = 64
CHUNK = 512


def _body(total, vocab, inp_hbm, w2s_hbm, emb_hbm, out_hbm,
          idx_v, rows_v, w2s_v, emb_v):
    wid = lax.axis_index("s") * NC + lax.axis_index("c")
    per_w = total // NW
    iters = per_w // CHUNK
    woff = wid * per_w

    pltpu.sync_copy(w2s_hbm, w2s_v)
    pltpu.sync_copy(emb_hbm, emb_v)

    iota64 = jnp.arange(L, dtype=jnp.int32) * EMB_DIM

    def chunk_body(i, carry):
        off = woff + i * CHUNK
        pltpu.sync_copy(inp_hbm.at[pl.ds(off, CHUNK)], idx_v)

        def group_body(g, c2):
            widx = idx_v[pl.ds(g * L, L)]
            cls = plsc.load_gather(w2s_v, [widx])
            ebase = cls * EMB_DIM
            pbase = g * (L * EMB_DIM) + iota64
            for d in range(EMB_DIM):
                vals = plsc.load_gather(emb_v, [ebase + d])
                plsc.store_scatter(rows_v, [pbase + d], vals)
            return c2

        lax.fori_loop(0, CHUNK // L, group_body, 0)
        pltpu.sync_copy(rows_v, out_hbm.at[pl.ds(off * EMB_DIM, CHUNK * EMB_DIM)])
        return carry

    lax.fori_loop(0, iters, chunk_body, 0)


@jax.jit
def _impl(inp, w2s, emb):
    batch, hist = inp.shape
    total = batch * hist
    vocab = w2s.shape[0]
    inp_flat = inp.astype(jnp.int32).reshape(total)
    emb_flat = emb.reshape(-1)

    mesh = plsc.VectorSubcoreMesh(core_axis_name="c", subcore_axis_name="s")

    out = pl.kernel(
        functools.partial(_body, total, vocab),
        out_type=jax.ShapeDtypeStruct((total * EMB_DIM,), jnp.float32),
        mesh=mesh,
        compiler_params=pltpu.CompilerParams(needs_layout_passes=False),
        scratch_types=[
            pltpu.VMEM((CHUNK,), jnp.int32),
            pltpu.VMEM((CHUNK * EMB_DIM,), jnp.float32),
            pltpu.VMEM((vocab,), jnp.int32),
            pltpu.VMEM((emb.size,), jnp.float32),
        ],
    )(inp_flat, w2s.astype(jnp.int32), emb_flat)

    return out.reshape(batch, hist, EMB_DIM)


def kernel(input, word2syllable, embedding):
    return _impl(input, word2syllable, embedding)


# speedup vs baseline: 14.4291x; 3.1697x over previous
import functools

import jax
import jax.numpy as jnp
from jax import lax
from jax.experimental import pallas as pl
from jax.experimental.pallas import tpu as pltpu
from jax.experimental.pallas import tpu_sc as plsc

NC = 2
NS = 16
NW = NC * NS
L = 16

EMB_DIM = 64
NCLS = 50
NPAIR = NCLS * NCLS
NPAIR_PAD = 2560
CHUNK = 512
CPAIR = CHUNK // 2


def _pair_body(emb_hbm, pair_hbm, emb_v, row_v):
    wid = lax.axis_index("s") * NC + lax.axis_index("c")
    n = NPAIR_PAD // NW
    base = wid * n
    pltpu.sync_copy(emb_hbm, emb_v)
    iota = jnp.arange(L, dtype=jnp.int32)
    for k in range(n):
        r = jnp.minimum(base + k, NPAIR - 1)
        c0 = r // NCLS
        c1 = r % NCLS
        for h in range(EMB_DIM // L):
            v0 = plsc.load_gather(emb_v, [c0 * EMB_DIM + h * L + iota])
            v1 = plsc.load_gather(emb_v, [c1 * EMB_DIM + h * L + iota])
            row_v[k, pl.ds(h * L, L)] = v0
            row_v[k, pl.ds(EMB_DIM + h * L, L)] = v1
    pltpu.sync_copy(row_v, pair_hbm.at[pl.ds(base, n)])


def _gather_body(total, inp_hbm, w2s_hbm, pair_hbm, out_hbm,
                 w2s_v, idx_v, pidx_v, rows_v, sem):
    wid = lax.axis_index("s") * NC + lax.axis_index("c")
    per_w = total // NW
    iters = per_w // CHUNK
    woff = wid * per_w

    pltpu.sync_copy(w2s_hbm, w2s_v)
    iota2 = jnp.arange(L, dtype=jnp.int32) * 2

    def chunk_body(i, carry):
        off = pl.multiple_of(woff + i * CHUNK, CHUNK)
        pltpu.sync_copy(inp_hbm.at[pl.ds(off, CHUNK)], idx_v)
        for g in range(CHUNK // (2 * L)):
            b = g * 2 * L
            evens = plsc.load_gather(idx_v, [b + iota2])
            odds = plsc.load_gather(idx_v, [b + iota2 + 1])
            c0 = plsc.load_gather(w2s_v, [evens])
            c1 = plsc.load_gather(w2s_v, [odds])
            pidx_v[pl.ds(g * L, L)] = c0 * NCLS + c1
        for j in range(CPAIR // 128):
            pltpu.async_copy(
                pair_hbm.at[pidx_v.at[pl.ds(j * 128, 128)]],
                rows_v.at[pl.ds(j * 128, 128)],
                sem,
            ).wait()
        poff = pl.multiple_of(off // 2, CPAIR)
        pltpu.sync_copy(rows_v, out_hbm.at[pl.ds(poff, CPAIR)])
        return carry

    lax.fori_loop(0, iters, chunk_body, 0)


@jax.jit
def _impl(inp, w2s, emb):
    batch, hist = inp.shape
    total = batch * hist
    inp_flat = inp.astype(jnp.int32).reshape(total)

    mesh = plsc.VectorSubcoreMesh(core_axis_name="c", subcore_axis_name="s")
    params = pltpu.CompilerParams(needs_layout_passes=False)

    pair = pl.kernel(
        _pair_body,
        out_type=jax.ShapeDtypeStruct((NPAIR_PAD, 2 * EMB_DIM), jnp.float32),
        mesh=mesh,
        compiler_params=params,
        scratch_types=[
            pltpu.VMEM((NCLS * EMB_DIM,), jnp.float32),
            pltpu.VMEM((NPAIR_PAD // NW, 2 * EMB_DIM), jnp.float32),
        ],
    )(emb.reshape(-1))

    out = pl.kernel(
        functools.partial(_gather_body, total),
        out_type=jax.ShapeDtypeStruct((total // 2, 2 * EMB_DIM), jnp.float32),
        mesh=mesh,
        compiler_params=params,
        scratch_types=[
            pltpu.VMEM((w2s.shape[0],), jnp.int32),
            pltpu.VMEM((CHUNK,), jnp.int32),
            pltpu.VMEM((CPAIR,), jnp.int32),
            pltpu.VMEM((CPAIR, 2 * EMB_DIM), jnp.float32),
            pltpu.SemaphoreType.DMA,
        ],
    )(inp_flat, w2s.astype(jnp.int32), pair)

    return out.reshape(batch, hist, EMB_DIM)


def kernel(input, word2syllable, embedding):
    return _impl(input, word2syllable, embedding)
